# baseline (speedup 1.0000x reference)
import functools

import jax
import jax.numpy as jnp
from jax import lax
from jax.experimental import pallas as pl
from jax.experimental.pallas import tpu as pltpu
from jax.experimental.pallas import tpu_sc as plsc

B, T, BS, EMBED = 64, 24, 8, 1024
N = EMBED // 4

NC, NS = 2, 16
NW = NC * NS
ROWS = B * T * BS
RPW = ROWS // NW
NCHUNK = 8
CHUNK = RPW // NCHUNK
BTPW = RPW // BS
BTPC = CHUNK // BS


def _sc_body(tok_hbm, ts_hbm, ch_hbm, pe_hbm, mt_hbm, out_hbm,
             ts_v, me_v, pe_v, ch_v, buf0, buf1,
             sem_g, sin0, sin1, sout0, sout1):
    w = lax.axis_index("c") * NS + lax.axis_index("s")
    row0 = w * RPW
    bt0 = w * BTPW

    pltpu.sync_copy(ch_hbm, ch_v)
    pltpu.sync_copy(pe_hbm, pe_v)
    pltpu.sync_copy(ts_hbm.at[pl.ds(bt0, BTPW)], ts_v)
    pltpu.async_copy(mt_hbm.at[ts_v], me_v, sem_g).wait()

    bufs = (buf0, buf1)
    sins = (sin0, sin1)
    souts = (sout0, sout1)

    def in_copy(c, buf, sem):
        return pltpu.make_async_copy(
            tok_hbm.at[pl.ds(row0 + c * CHUNK, CHUNK), :], buf, sem)

    def out_copy(c, buf, sem):
        return pltpu.make_async_copy(
            buf, out_hbm.at[pl.ds(row0 + c * CHUNK, CHUNK), :], sem)

    in_copy(0, bufs[0], sins[0]).start()

    for c in range(NCHUNK):
        k = c % 2
        cur = bufs[k]
        in_copy(c, cur, sins[k]).wait()
        if c >= 1:
            out_copy(c - 1, bufs[1 - k], souts[1 - k]).wait()
        if c + 1 < NCHUNK:
            in_copy(c + 1, bufs[1 - k], sins[1 - k]).start()

        def row_body(rr, _):
            s = rr & 7
            g = rr >> 3
            m = g + BTPC * c
            tt = m + (bt0 % T)
            t = jnp.where(tt >= T, tt - T, tt)
            for j in range(N // 16):
                o = j * 16
                cur[rr, pl.ds(o, 16)] = (
                    cur[rr, pl.ds(o, 16)] + ch_v[s, pl.ds(o, 16)])
                cur[rr, pl.ds(N + o, 16)] = (
                    cur[rr, pl.ds(N + o, 16)] + pe_v[t, pl.ds(o, 16)])
                cur[rr, pl.ds(2 * N + o, 16)] = (
                    cur[rr, pl.ds(2 * N + o, 16)] + me_v[m, pl.ds(o, 16)])
            return _

        lax.fori_loop(0, CHUNK, row_body, None)
        out_copy(c, cur, souts[k]).start()

    out_copy(NCHUNK - 1, bufs[(NCHUNK - 1) % 2], souts[(NCHUNK - 1) % 2]).wait()


def kernel(modality_tokens, timestamps, channel_embed, pos_embed, month_table):
    tok3 = modality_tokens.reshape(ROWS, EMBED)
    ts_flat = timestamps.astype(jnp.int32).reshape(B * T)

    mesh = plsc.VectorSubcoreMesh(core_axis_name="c", subcore_axis_name="s")
    sc = functools.partial(
        pl.kernel,
        mesh=mesh,
        out_type=jax.ShapeDtypeStruct((ROWS, EMBED), jnp.float32),
        scratch_types=[
            pltpu.VMEM((BTPW,), jnp.int32),
            pltpu.VMEM((BTPW, N), jnp.float32),
            pltpu.VMEM((T, N), jnp.float32),
            pltpu.VMEM((BS, N), jnp.float32),
            pltpu.VMEM((CHUNK, EMBED), jnp.float32),
            pltpu.VMEM((CHUNK, EMBED), jnp.float32),
            pltpu.SemaphoreType.DMA,
            pltpu.SemaphoreType.DMA,
            pltpu.SemaphoreType.DMA,
            pltpu.SemaphoreType.DMA,
            pltpu.SemaphoreType.DMA,
        ],
    )(_sc_body)
    out3 = sc(tok3, ts_flat, channel_embed, pos_embed, month_table)
    return out3.reshape(B, T, BS, EMBED)


# speedup vs baseline: 2.2794x; 2.2794x over previous
import functools

import jax
import jax.numpy as jnp
from jax import lax
from jax.experimental import pallas as pl
from jax.experimental.pallas import tpu as pltpu
from jax.experimental.pallas import tpu_sc as plsc

B, T, BS, EMBED = 64, 24, 8, 1024
N = EMBED // 4

NC, NS = 2, 16
NW = NC * NS
BT = B * T
BTPW = BT // NW

BBLK = 16


def _sc_gather_body(ts_hbm, mt_hbm, out_hbm, ts_v, me_v, sem_g):
    w = lax.axis_index("c") * NS + lax.axis_index("s")
    bt0 = w * BTPW
    pltpu.sync_copy(ts_hbm.at[pl.ds(bt0, BTPW)], ts_v)
    pltpu.async_copy(mt_hbm.at[ts_v], me_v, sem_g).wait()
    pltpu.sync_copy(me_v, out_hbm.at[pl.ds(bt0, BTPW), :])


def _tc_body(tok_ref, ch_ref, pe_ref, me_ref, out_ref):
    ch = ch_ref[...]
    for bi in range(BBLK):
        for t in range(T):
            me = me_ref[bi, t]
            pe = pe_ref[t, :]
            tok = tok_ref[bi, t]
            out_ref[bi, t] = jnp.concatenate(
                [
                    tok[:, :N] + ch,
                    tok[:, N:2 * N] + pe[None, :],
                    tok[:, 2 * N:3 * N] + me[None, :],
                    tok[:, 3 * N:],
                ],
                axis=-1,
            )


def kernel(modality_tokens, timestamps, channel_embed, pos_embed, month_table):
    ts_flat = timestamps.astype(jnp.int32).reshape(BT)

    mesh = plsc.VectorSubcoreMesh(core_axis_name="c", subcore_axis_name="s")
    me_rows = functools.partial(
        pl.kernel,
        mesh=mesh,
        out_type=jax.ShapeDtypeStruct((BT, N), jnp.float32),
        scratch_types=[
            pltpu.VMEM((BTPW,), jnp.int32),
            pltpu.VMEM((BTPW, N), jnp.float32),
            pltpu.SemaphoreType.DMA,
        ],
    )(_sc_gather_body)(ts_flat, month_table)
    me4 = me_rows.reshape(B, T, N)

    return pl.pallas_call(
        _tc_body,
        grid=(B // BBLK,),
        in_specs=[
            pl.BlockSpec((BBLK, T, BS, EMBED), lambda b: (b, 0, 0, 0)),
            pl.BlockSpec((BS, N), lambda b: (0, 0)),
            pl.BlockSpec((T, N), lambda b: (0, 0)),
            pl.BlockSpec((BBLK, T, N), lambda b: (b, 0, 0)),
        ],
        out_specs=pl.BlockSpec((BBLK, T, BS, EMBED), lambda b: (b, 0, 0, 0)),
        out_shape=jax.ShapeDtypeStruct((B, T, BS, EMBED), jnp.float32),
    )(modality_tokens, channel_embed, pos_embed, me4)
